# baseline (speedup 1.0000x reference)
import functools

import jax
import jax.numpy as jnp
from jax.experimental import pallas as pl
from jax.experimental.pallas import tpu as pltpu

KNN_K = 16
TILE_Q = 256
LANES = 128


def _topk_cols(m, k):
    rows, width = m.shape
    lane = jax.lax.broadcasted_iota(jnp.int32, (rows, width), 1)
    cols = []
    for _ in range(k):
        a = jnp.argmin(m, axis=1, keepdims=True).astype(jnp.int32)
        cols.append(a)
        m = jnp.where(lane == a, jnp.inf, m)
    return jnp.concatenate(cols, axis=1)


def _knn_body(r_ref, q_ref, d_ref, i_ref, r2_ref, *, nref, dim, tq, k):
    q = q_ref[0]
    r = r_ref[0]

    q2 = jnp.sum(q * q, axis=1, keepdims=True)

    @pl.when(pl.program_id(1) == 0)
    def _():
        r2 = jnp.sum(r * r, axis=1, keepdims=True)
        r2_ref[...] = r2.reshape(1, nref)

    r2row = r2_ref[...]
    cross2 = jax.lax.dot_general(
        -2.0 * q, r, (((1,), (1,)), ((), ())),
        preferred_element_type=jnp.float32)
    d2 = (q2 + r2row) + cross2

    s = LANES
    rsz = nref // s
    x = d2.reshape(tq, rsz, s)

    m1 = jnp.min(x, axis=1)
    s1 = _topk_cols(m1, k)
    idx1 = jnp.broadcast_to(s1[:, None, :], (tq, rsz, k))
    c0 = jnp.take_along_axis(x, idx1, axis=2,
                             mode="promise_in_bounds")
    c1 = jnp.swapaxes(c0, 1, 2)
    g1 = (jax.lax.broadcasted_iota(jnp.int32, (tq, k, rsz), 2) * s
          + jnp.broadcast_to(s1[:, :, None], (tq, k, rsz)))

    big = jnp.int32(jnp.iinfo(jnp.int32).max)
    dcols, icols = [], []
    for _ in range(k):
        m2 = jnp.min(c1, axis=1)
        v = jnp.min(m2, axis=1, keepdims=True)
        hit = c1 == v[:, :, None]
        gmask = jnp.where(hit, g1, big)
        gi = jnp.min(jnp.min(gmask, axis=2), axis=1, keepdims=True)
        dcols.append(v)
        icols.append(gi)
        c1 = jnp.where(hit, jnp.inf, c1)
    dist = jnp.sqrt(jnp.maximum(jnp.concatenate(dcols, axis=1), 0.0))
    d_ref[0] = dist
    i_ref[0] = jnp.concatenate(icols, axis=1)


def kernel(ref, query):
    b, nref, dim = ref.shape
    _, nq, _ = query.shape
    tq = min(TILE_Q, nq)
    k = KNN_K
    grid = (b, nq // tq)
    body = functools.partial(_knn_body, nref=nref, dim=dim, tq=tq, k=k)
    dist, idx = pl.pallas_call(
        body,
        grid=grid,
        in_specs=[
            pl.BlockSpec((1, nref, dim), lambda bi, qi: (bi, 0, 0)),
            pl.BlockSpec((1, tq, dim), lambda bi, qi: (bi, qi, 0)),
        ],
        out_specs=[
            pl.BlockSpec((1, tq, k), lambda bi, qi: (bi, qi, 0)),
            pl.BlockSpec((1, tq, k), lambda bi, qi: (bi, qi, 0)),
        ],
        out_shape=[
            jax.ShapeDtypeStruct((b, nq, k), jnp.float32),
            jax.ShapeDtypeStruct((b, nq, k), jnp.int32),
        ],
        scratch_shapes=[pltpu.VMEM((1, nref), jnp.float32)],
        compiler_params=pltpu.CompilerParams(
            dimension_semantics=("parallel", "arbitrary")),
    )(ref, query)
    return (dist, idx.astype(jnp.int64))


# speedup vs baseline: 1.0781x; 1.0781x over previous
import functools

import jax
import jax.numpy as jnp
from jax.experimental import pallas as pl
from jax.experimental.pallas import tpu as pltpu

KNN_K = 16
TILE_Q = 256
LANES = 128


def _topk_cols(m, k):
    rows, width = m.shape
    lane = jax.lax.broadcasted_iota(jnp.int32, (rows, width), 1)
    cols = []
    for _ in range(k):
        a = jnp.argmin(m, axis=1, keepdims=True).astype(jnp.int32)
        cols.append(a)
        m = jnp.where(lane == a, jnp.inf, m)
    return jnp.concatenate(cols, axis=1)


def _knn_body(r_ref, q_ref, d_ref, i_ref, *, nref, dim, tq, k):
    q = q_ref[0]
    r = r_ref[0]

    q2 = jnp.sum(q * q, axis=1, keepdims=True)
    r2 = jnp.sum(r * r, axis=1, keepdims=True)
    r2row = r2.reshape(1, nref)
    cross2 = jax.lax.dot_general(
        -2.0 * q, r, (((1,), (1,)), ((), ())),
        preferred_element_type=jnp.float32)
    d2 = (q2 + r2row) + cross2

    s = LANES
    rsz = nref // s
    x = d2.reshape(tq, rsz, s)

    m1 = jnp.min(x, axis=1)
    s1 = _topk_cols(m1, k)
    idx1 = jnp.broadcast_to(s1[:, None, :], (tq, rsz, k))
    c0 = jnp.take_along_axis(x, idx1, axis=2,
                             mode="promise_in_bounds")
    c1 = jnp.swapaxes(c0, 1, 2)
    g1 = (jax.lax.broadcasted_iota(jnp.int32, (tq, k, rsz), 2) * s
          + jnp.broadcast_to(s1[:, :, None], (tq, k, rsz)))

    big = jnp.int32(jnp.iinfo(jnp.int32).max)
    dcols, icols = [], []
    for _ in range(k):
        m2 = jnp.min(c1, axis=1)
        v = jnp.min(m2, axis=1, keepdims=True)
        hit = c1 == v[:, :, None]
        gmask = jnp.where(hit, g1, big)
        gi = jnp.min(jnp.min(gmask, axis=2), axis=1, keepdims=True)
        dcols.append(v)
        icols.append(gi)
        c1 = jnp.where(hit, jnp.inf, c1)
    dist = jnp.sqrt(jnp.maximum(jnp.concatenate(dcols, axis=1), 0.0))
    d_ref[0] = dist
    i_ref[0] = jnp.concatenate(icols, axis=1)


def kernel(ref, query):
    b, nref, dim = ref.shape
    _, nq, _ = query.shape
    tq = min(TILE_Q, nq)
    k = KNN_K
    grid = (b, nq // tq)
    body = functools.partial(_knn_body, nref=nref, dim=dim, tq=tq, k=k)
    dist, idx = pl.pallas_call(
        body,
        grid=grid,
        in_specs=[
            pl.BlockSpec((1, nref, dim), lambda bi, qi: (bi, 0, 0)),
            pl.BlockSpec((1, tq, dim), lambda bi, qi: (bi, qi, 0)),
        ],
        out_specs=[
            pl.BlockSpec((1, tq, k), lambda bi, qi: (bi, qi, 0)),
            pl.BlockSpec((1, tq, k), lambda bi, qi: (bi, qi, 0)),
        ],
        out_shape=[
            jax.ShapeDtypeStruct((b, nq, k), jnp.float32),
            jax.ShapeDtypeStruct((b, nq, k), jnp.int32),
        ],
        compiler_params=pltpu.CompilerParams(
            dimension_semantics=("parallel", "parallel")),
    )(ref, query)
    return (dist, idx.astype(jnp.int64))


# speedup vs baseline: 1.5706x; 1.4568x over previous
import functools

import jax
import jax.numpy as jnp
from jax.experimental import pallas as pl
from jax.experimental.pallas import tpu as pltpu

KNN_K = 16
TILE_Q = 256
LANES = 128


def _topk_cols(m, k):
    rows, width = m.shape
    lane = jax.lax.broadcasted_iota(jnp.int32, (rows, width), 1)
    cols = []
    for _ in range(k):
        a = jnp.argmin(m, axis=1, keepdims=True).astype(jnp.int32)
        cols.append(a)
        m = jnp.where(lane == a, jnp.inf, m)
    return jnp.concatenate(cols, axis=1)


def _knn_body(r_ref, q_ref, d_ref, i_ref, *, nref, dim, tq, k):
    q = q_ref[0]
    r = r_ref[0]

    q2 = jnp.sum(q * q, axis=1, keepdims=True)
    r2 = jnp.sum(r * r, axis=1, keepdims=True)
    r2row = r2.reshape(1, nref)
    cross2 = jax.lax.dot_general(
        -2.0 * q, r, (((1,), (1,)), ((), ())),
        preferred_element_type=jnp.float32)
    d2 = (q2 + r2row) + cross2

    s = LANES
    rsz = nref // s
    x = d2.reshape(tq, rsz, s)

    m1 = jnp.min(x, axis=1)
    s1 = _topk_cols(m1, k)
    idx1 = jnp.broadcast_to(s1[:, None, :], (tq, rsz, k))
    c0 = jnp.take_along_axis(x, idx1, axis=2,
                             mode="promise_in_bounds")
    c1 = jnp.swapaxes(c0, 1, 2)
    g1 = (jax.lax.broadcasted_iota(jnp.int32, (tq, k, rsz), 2) * s
          + jnp.broadcast_to(s1[:, :, None], (tq, k, rsz)))

    big = jnp.int32(jnp.iinfo(jnp.int32).max)
    dcols, icols = [], []
    for _ in range(k):
        m2 = jnp.min(c1, axis=1)
        v = jnp.min(m2, axis=1, keepdims=True)
        hit = c1 == v[:, :, None]
        gmask = jnp.where(hit, g1, big)
        gi = jnp.min(jnp.min(gmask, axis=1), axis=1, keepdims=True)
        dcols.append(v)
        icols.append(gi)
        c1 = jnp.where(hit, jnp.inf, c1)
    dist = jnp.sqrt(jnp.maximum(jnp.concatenate(dcols, axis=1), 0.0))
    d_ref[0] = dist
    i_ref[0] = jnp.concatenate(icols, axis=1)


def kernel(ref, query):
    b, nref, dim = ref.shape
    _, nq, _ = query.shape
    tq = min(TILE_Q, nq)
    k = KNN_K
    grid = (b, nq // tq)
    body = functools.partial(_knn_body, nref=nref, dim=dim, tq=tq, k=k)
    dist, idx = pl.pallas_call(
        body,
        grid=grid,
        in_specs=[
            pl.BlockSpec((1, nref, dim), lambda bi, qi: (bi, 0, 0)),
            pl.BlockSpec((1, tq, dim), lambda bi, qi: (bi, qi, 0)),
        ],
        out_specs=[
            pl.BlockSpec((1, tq, k), lambda bi, qi: (bi, qi, 0)),
            pl.BlockSpec((1, tq, k), lambda bi, qi: (bi, qi, 0)),
        ],
        out_shape=[
            jax.ShapeDtypeStruct((b, nq, k), jnp.float32),
            jax.ShapeDtypeStruct((b, nq, k), jnp.int32),
        ],
        compiler_params=pltpu.CompilerParams(
            dimension_semantics=("parallel", "parallel")),
    )(ref, query)
    return (dist, idx.astype(jnp.int64))


# speedup vs baseline: 1.6213x; 1.0323x over previous
import functools

import jax
import jax.numpy as jnp
from jax.experimental import pallas as pl
from jax.experimental.pallas import tpu as pltpu

KNN_K = 16
TILE_Q = 256
LANES = 128


def _topk_cols(m, k):
    rows, width = m.shape
    lane = jax.lax.broadcasted_iota(jnp.int32, (rows, width), 1)
    cols = []
    for _ in range(k):
        a = jnp.argmin(m, axis=1, keepdims=True).astype(jnp.int32)
        cols.append(a)
        m = jnp.where(lane == a, jnp.inf, m)
    return jnp.concatenate(cols, axis=1)


def _knn_body(r_ref, q_ref, q2_ref, r2_ref, d_ref, i_ref, *, nref, dim, tq, k):
    q = q_ref[0]
    r = r_ref[0]

    q2 = q2_ref[0]
    r2row = r2_ref[0, 0:1, :]
    cross2 = jax.lax.dot_general(
        -2.0 * q, r, (((1,), (1,)), ((), ())),
        preferred_element_type=jnp.float32)
    d2 = (q2 + r2row) + cross2

    s = LANES
    rsz = nref // s
    x = d2.reshape(tq, rsz, s)

    m1 = jnp.min(x, axis=1)
    s1 = _topk_cols(m1, k)
    idx1 = jnp.broadcast_to(s1[:, None, :], (tq, rsz, k))
    c0 = jnp.take_along_axis(x, idx1, axis=2,
                             mode="promise_in_bounds")
    c1 = jnp.swapaxes(c0, 1, 2)
    g1 = (jax.lax.broadcasted_iota(jnp.int32, (tq, k, rsz), 2) * s
          + jnp.broadcast_to(s1[:, :, None], (tq, k, rsz)))

    big = jnp.int32(jnp.iinfo(jnp.int32).max)
    dcols, icols = [], []
    for _ in range(k):
        m2 = jnp.min(c1, axis=1)
        v = jnp.min(m2, axis=1, keepdims=True)
        hit = c1 == v[:, :, None]
        gmask = jnp.where(hit, g1, big)
        gi = jnp.min(jnp.min(gmask, axis=1), axis=1, keepdims=True)
        dcols.append(v)
        icols.append(gi)
        c1 = jnp.where(hit, jnp.inf, c1)
    dist = jnp.sqrt(jnp.maximum(jnp.concatenate(dcols, axis=1), 0.0))
    d_ref[0] = dist
    i_ref[0] = jnp.concatenate(icols, axis=1)


def kernel(ref, query):
    b, nref, dim = ref.shape
    _, nq, _ = query.shape
    tq = min(TILE_Q, nq)
    k = KNN_K
    grid = (b, nq // tq)
    q2 = jnp.sum(query * query, axis=-1)[:, :, None]
    r2 = jnp.broadcast_to(jnp.sum(ref * ref, axis=-1)[:, None, :],
                          (b, 8, nref))
    body = functools.partial(_knn_body, nref=nref, dim=dim, tq=tq, k=k)
    dist, idx = pl.pallas_call(
        body,
        grid=grid,
        in_specs=[
            pl.BlockSpec((1, nref, dim), lambda bi, qi: (bi, 0, 0)),
            pl.BlockSpec((1, tq, dim), lambda bi, qi: (bi, qi, 0)),
            pl.BlockSpec((1, tq, 1), lambda bi, qi: (bi, qi, 0)),
            pl.BlockSpec((1, 8, nref), lambda bi, qi: (bi, 0, 0)),
        ],
        out_specs=[
            pl.BlockSpec((1, tq, k), lambda bi, qi: (bi, qi, 0)),
            pl.BlockSpec((1, tq, k), lambda bi, qi: (bi, qi, 0)),
        ],
        out_shape=[
            jax.ShapeDtypeStruct((b, nq, k), jnp.float32),
            jax.ShapeDtypeStruct((b, nq, k), jnp.int32),
        ],
        compiler_params=pltpu.CompilerParams(
            dimension_semantics=("parallel", "parallel")),
    )(ref, query, q2, r2)
    return (dist, idx.astype(jnp.int64))
